# baseline (speedup 1.0000x reference)
import functools

import jax
import jax.numpy as jnp
from jax import lax
from jax.experimental import pallas as pl
from jax.experimental.pallas import tpu as pltpu
from jax.experimental.pallas import tpu_sc as plsc

_N = 2048
_E = 8192
_ROWS = 8
_NB = _N // _ROWS
_W = 32
_WB = 128
_NC = 2
_NS = 16
_NW = _NC * _NS
_OPW = _E // _NW


def _row_cumsum(x):
    n = x.shape[-1]
    s = 1
    while s < n:
        shifted = jnp.concatenate([jnp.zeros_like(x[:, :s]), x[:, :-s]], axis=-1)
        x = x + shifted
        s *= 2
    return x


def _tca_kernel(adj_ref, table_ref):
    block = adj_ref[...]
    mask = (block > 0.0).astype(jnp.float32)
    cum = _row_cumsum(mask)
    icum = cum.astype(jnp.int32)
    kiota = jax.lax.broadcasted_iota(jnp.int32, (_W, _N), 0) + 1

    cvs = []
    for r in range(_ROWS):
        vrow = block[r, :]
        hit = icum[r, :][None, :] == kiota
        c = jnp.sum(jnp.where(hit, vrow[None, :], 0.0), axis=1)
        cvs.append(c[:, None])

    tot = cum[:, _N - 1:_N]
    incl = tot
    s = 1
    while s < _ROWS:
        incl = incl + jnp.concatenate(
            [jnp.zeros_like(incl[:s]), incl[:-s]], axis=0)
        s *= 2
    rp = (incl - tot).astype(jnp.int32)

    kio = jax.lax.broadcasted_iota(jnp.int32, (_W, 1), 0)
    pos_parts = []
    for r in range(_ROWS):
        pos_parts.append(jnp.broadcast_to(rp[r:r + 1, :], (_W, 1)) + kio)
    pos256 = jnp.concatenate(pos_parts, axis=0)
    val256 = jnp.concatenate(cvs, axis=0)

    jio = jax.lax.broadcasted_iota(jnp.int32, (_ROWS * _W, _WB), 1)
    bc = jnp.sum(jnp.where(pos256 == jio, val256, 0.0), axis=0)
    table_ref[...] = bc[None, None, :]


def _tca_call(adj):
    return pl.pallas_call(
        _tca_kernel,
        grid=(_NB,),
        in_specs=[pl.BlockSpec((_ROWS, _N), lambda i: (i, 0))],
        out_specs=pl.BlockSpec((1, 1, _WB), lambda i: (i, 0, 0)),
        out_shape=jax.ShapeDtypeStruct((_NB, 1, _WB), jnp.float32),
        compiler_params=pltpu.CompilerParams(
            dimension_semantics=("parallel",),
        ),
    )(adj)


def _tcb_kernel(table_ref, src_ref):
    e = (jax.lax.broadcasted_iota(jnp.int32, (_E // 128, 128), 0) * 128
         + jax.lax.broadcasted_iota(jnp.int32, (_E // 128, 128), 1))
    sblk = jnp.zeros_like(e)
    pe = jnp.zeros_like(e)
    incl = jnp.zeros((1, 1), jnp.int32)
    for k in range(_NB):
        row = table_ref[k, :, :]
        cnt_k = jnp.sum((row > 0.0).astype(jnp.int32), axis=1,
                        keepdims=True)
        incl = incl + cnt_k
        hit = (jnp.broadcast_to(incl, e.shape) <= e).astype(jnp.int32)
        sblk = sblk + hit
        pe = pe + jnp.broadcast_to(cnt_k, e.shape) * hit
    src_ref[...] = sblk * _WB + (e - pe)


def _tcb_call(table):
    return pl.pallas_call(
        _tcb_kernel,
        in_specs=[pl.BlockSpec((_NB, 1, _WB), lambda: (0, 0, 0))],
        out_specs=pl.BlockSpec((_E // 128, 128), lambda: (0, 0)),
        out_shape=jax.ShapeDtypeStruct((_E // 128, 128), jnp.int32),
    )(table)


_sc_mesh = plsc.VectorSubcoreMesh(
    core_axis_name="c", subcore_axis_name="s",
    num_cores=_NC, num_subcores=_NS,
)


def _scc_body(table, src, out, sidx, gbuf):
    w = lax.axis_index("s") * _NC + lax.axis_index("c")
    base = w * _OPW
    pltpu.sync_copy(src.at[pl.ds(base, _OPW)], sidx)
    for blk in range(_OPW // 128):
        pltpu.sync_copy(
            table.at[sidx.at[pl.ds(blk * 128, 128)]],
            gbuf.at[pl.ds(blk * 128, 128)],
        )
    pltpu.sync_copy(gbuf, out.at[pl.ds(base, _OPW)])


_scc_call = functools.partial(
    pl.kernel,
    out_type=jax.ShapeDtypeStruct((_E,), jnp.float32),
    mesh=_sc_mesh,
    scratch_types=[
        pltpu.VMEM((_OPW,), jnp.int32),
        pltpu.VMEM((_OPW,), jnp.float32),
    ],
)(_scc_body)


def kernel(atoms, adjacency_map):
    del atoms
    table = _tca_call(adjacency_map)
    src = _tcb_call(table)
    out = _scc_call(table.reshape(-1), src.reshape(-1))
    return out[:, None]


# speedup vs baseline: 1.3094x; 1.3094x over previous
import jax
import jax.numpy as jnp
from jax.experimental import pallas as pl
from jax.experimental.pallas import tpu as pltpu

_N = 2048
_E = 8192
_ROWS = 8
_W = 32


def _row_cumsum(x):
    n = x.shape[-1]
    s = 1
    while s < n:
        shifted = jnp.concatenate([jnp.zeros_like(x[:, :s]), x[:, :-s]], axis=-1)
        x = x + shifted
        s *= 2
    return x


def _compact_kernel(adj_ref, out_ref, off_ref):
    i = pl.program_id(0)

    @pl.when(i == 0)
    def _init():
        off_ref[0] = 0

    block = adj_ref[...]
    mask = (block > 0.0).astype(jnp.float32)
    cum = _row_cumsum(mask)
    kiota = jax.lax.broadcasted_iota(jnp.int32, (_W, _N), 0) + 1
    icum = cum.astype(jnp.int32)

    off = off_ref[0]
    for r in range(_ROWS):
        vrow = block[r, :]
        hit = icum[r, :][None, :] == kiota
        c = jnp.sum(jnp.where(hit, vrow[None, :], 0.0), axis=1)
        out_ref[pl.ds(off, _W), :] = c[:, None]
        cnt = jnp.sum(mask[r, :]).astype(jnp.int32)
        off = off + cnt
    off_ref[0] = off


def kernel(atoms, adjacency_map):
    del atoms
    padded = pl.pallas_call(
        _compact_kernel,
        grid=(_N // _ROWS,),
        in_specs=[pl.BlockSpec((_ROWS, _N), lambda i: (i, 0))],
        out_specs=pl.BlockSpec((_E + _W, 1), lambda i: (0, 0)),
        out_shape=jax.ShapeDtypeStruct((_E + _W, 1), jnp.float32),
        scratch_shapes=[pltpu.SMEM((1,), jnp.int32)],
        compiler_params=pltpu.CompilerParams(
            dimension_semantics=("arbitrary",),
        ),
    )(adjacency_map)
    return padded[:_E]


# speedup vs baseline: 3.0340x; 2.3170x over previous
import jax
import jax.numpy as jnp
from jax.experimental import pallas as pl
from jax.experimental.pallas import tpu as pltpu

_N = 2048
_E = 8192
_ROWS = 32
_W = 24


def _row_cumsum(x):
    n = x.shape[-1]
    s = 1
    while s < n:
        shifted = jnp.concatenate([jnp.zeros_like(x[:, :s]), x[:, :-s]], axis=-1)
        x = x + shifted
        s *= 2
    return x


def _compact_kernel(adj_ref, out_ref, off_ref):
    i = pl.program_id(0)

    @pl.when(i == 0)
    def _init():
        off_ref[0] = 0

    block = adj_ref[...]
    mask = (block > 0.0).astype(jnp.float32)
    cum = _row_cumsum(mask)
    kiota = jax.lax.broadcasted_iota(jnp.int32, (_W, _N), 0) + 1
    icum = cum.astype(jnp.int32)

    cnts = [jnp.sum(mask[r, :]).astype(jnp.int32) for r in range(_ROWS)]
    off = off_ref[0]
    offs = []
    for r in range(_ROWS):
        offs.append(off)
        off = off + cnts[r]
    off_ref[0] = off

    for r in range(_ROWS):
        vrow = block[r, :]
        hit = icum[r, :][None, :] == kiota
        c = jnp.sum(jnp.where(hit, vrow[None, :], 0.0), axis=1)
        out_ref[pl.ds(offs[r], _W), :] = c[:, None]


def kernel(atoms, adjacency_map):
    del atoms
    padded = pl.pallas_call(
        _compact_kernel,
        grid=(_N // _ROWS,),
        in_specs=[pl.BlockSpec((_ROWS, _N), lambda i: (i, 0))],
        out_specs=pl.BlockSpec((_E + _W, 1), lambda i: (0, 0)),
        out_shape=jax.ShapeDtypeStruct((_E + _W, 1), jnp.float32),
        scratch_shapes=[pltpu.SMEM((1,), jnp.int32)],
        compiler_params=pltpu.CompilerParams(
            dimension_semantics=("arbitrary",),
        ),
    )(adjacency_map)
    return padded[:_E]


# speedup vs baseline: 3.1868x; 1.0504x over previous
import jax
import jax.numpy as jnp
from jax.experimental import pallas as pl
from jax.experimental.pallas import tpu as pltpu

_N = 2048
_E = 8192
_ROWS = 64
_W = 24


def _row_cumsum(x):
    n = x.shape[-1]
    s = 1
    while s < n:
        shifted = jnp.concatenate([jnp.zeros_like(x[:, :s]), x[:, :-s]], axis=-1)
        x = x + shifted
        s *= 2
    return x


def _compact_kernel(adj_ref, out_ref, off_ref):
    i = pl.program_id(0)

    @pl.when(i == 0)
    def _init():
        off_ref[0] = 0

    block = adj_ref[...]
    mask = (block > 0.0).astype(jnp.float32)
    cum = _row_cumsum(mask)
    kiota = jax.lax.broadcasted_iota(jnp.int32, (_W, _N), 0) + 1
    icum = cum.astype(jnp.int32)

    cnts = [jnp.sum(mask[r, :]).astype(jnp.int32) for r in range(_ROWS)]
    off = off_ref[0]
    offs = []
    for r in range(_ROWS):
        offs.append(off)
        off = off + cnts[r]
    off_ref[0] = off

    for r in range(_ROWS):
        vrow = block[r, :]
        hit = icum[r, :][None, :] == kiota
        c = jnp.sum(jnp.where(hit, vrow[None, :], 0.0), axis=1)
        out_ref[pl.ds(offs[r], _W), :] = c[:, None]


def kernel(atoms, adjacency_map):
    del atoms
    padded = pl.pallas_call(
        _compact_kernel,
        grid=(_N // _ROWS,),
        in_specs=[pl.BlockSpec((_ROWS, _N), lambda i: (i, 0))],
        out_specs=pl.BlockSpec((_E + _W, 1), lambda i: (0, 0)),
        out_shape=jax.ShapeDtypeStruct((_E + _W, 1), jnp.float32),
        scratch_shapes=[pltpu.SMEM((1,), jnp.int32)],
        compiler_params=pltpu.CompilerParams(
            dimension_semantics=("arbitrary",),
        ),
    )(adjacency_map)
    return padded[:_E]
